# baseline (device time: 595429 ns/iter reference)
import jax
import jax.numpy as jnp
from jax import lax
from jax.experimental import pallas as pl
from jax.experimental.pallas import tpu as pltpu

N_DEV = 8
SCALE = 0.08838834764831843
BLK = 64
QT = 512


NC = 8
CH = 256
TOTC = 2 * NC

TREE_CHILDREN = [
    {0: [1], 1: [2, 5], 2: [3, 6], 5: [4], 6: [7]},
    {0: [3], 3: [7, 2], 7: [4, 6], 2: [1], 6: [5]},
    {0: [4], 4: [5, 7], 5: [1, 6], 7: [3], 6: [2]},
]


def _bcast_body(k_ref, v_ref, kout_ref, vout_ref, r_sems, s0_sems, s1_sems):
    my = lax.axis_index("i")

    def chunk_ref(kb, vb, c):
        base = kb if c < NC else vb
        return base.at[pl.ds((c % NC) * CH, CH), :]

    def desc(c, child, slot, from_input):
        kb = k_ref if from_input else kout_ref
        vb = v_ref if from_input else vout_ref
        sem = s0_sems if slot == 0 else s1_sems
        return pltpu.make_async_remote_copy(
            src_ref=chunk_ref(kb, vb, c),
            dst_ref=chunk_ref(kout_ref, vout_ref, c),
            send_sem=sem.at[c],
            recv_sem=r_sems.at[c],
            device_id=(child,),
            device_id_type=pl.DeviceIdType.MESH,
        )

    @pl.when(my == 0)
    def _():
        for c in range(TOTC):
            for slot, child in enumerate(TREE_CHILDREN[c % 3].get(0, [])):
                desc(c, child, slot, True).start()
        kout_ref[...] = k_ref[...]
        vout_ref[...] = v_ref[...]
        for c in range(TOTC):
            for slot, child in enumerate(TREE_CHILDREN[c % 3].get(0, [])):
                desc(c, child, slot, True).wait_send()

    for p in range(1, N_DEV):

        @pl.when(my == p)
        def _(p=p):
            sends = []
            for c in range(TOTC):
                desc(c, 0, 0, False).wait_recv()
                for slot, child in enumerate(TREE_CHILDREN[c % 3].get(p, [])):
                    desc(c, child, slot, False).start()
                    sends.append((c, child, slot))
            for c, child, slot in sends:
                desc(c, child, slot, False).wait_send()


def _attn_body(x_ref, wq_ref, k_ref, v_ref, wo_ref, out_ref):
    qt = pl.program_id(0)
    h = pl.program_id(1)

    xm = x_ref[0]
    q = jnp.dot(xm, wq_ref[...], preferred_element_type=jnp.float32)
    k = k_ref[...]
    s = lax.dot_general(
        q, k, (((1,), (1,)), ((), ())), preferred_element_type=jnp.float32
    )
    s = s * SCALE
    row = lax.broadcasted_iota(jnp.int32, s.shape, 0) + qt * QT
    col = lax.broadcasted_iota(jnp.int32, s.shape, 1)
    s = jnp.where((col // BLK) <= (row // BLK), s, -1e9)
    m = jnp.max(s, axis=1, keepdims=True)
    w = jnp.exp(s - m)
    w = w / jnp.sum(w, axis=1, keepdims=True)
    ctx = jnp.dot(w, v_ref[...], preferred_element_type=jnp.float32)
    contrib = jnp.dot(ctx, wo_ref[...], preferred_element_type=jnp.float32)

    @pl.when(h == 0)
    def _():
        out_ref[...] = contrib[None]

    @pl.when(h > 0)
    def _():
        out_ref[...] = out_ref[...] + contrib[None]


def kernel(x, Wq, K_ext, V_ext, Wo):
    B, Sq, Dm = x.shape
    _, Skv, Hq, Dh = K_ext.shape

    k2 = K_ext.reshape(Skv, Hq * Dh)
    v2 = V_ext.reshape(Skv, Hq * Dh)

    kfull, vfull = pl.pallas_call(
        _bcast_body,
        out_shape=[
            jax.ShapeDtypeStruct((Skv, Hq * Dh), jnp.float32),
            jax.ShapeDtypeStruct((Skv, Hq * Dh), jnp.float32),
        ],
        in_specs=[
            pl.BlockSpec(memory_space=pltpu.VMEM),
            pl.BlockSpec(memory_space=pltpu.VMEM),
        ],
        out_specs=[
            pl.BlockSpec(memory_space=pltpu.VMEM),
            pl.BlockSpec(memory_space=pltpu.VMEM),
        ],
        scratch_shapes=[
            pltpu.SemaphoreType.DMA((TOTC,)),
            pltpu.SemaphoreType.DMA((TOTC,)),
            pltpu.SemaphoreType.DMA((TOTC,)),
        ],
    )(k2, v2)

    n_qt = Sq // QT
    out = pl.pallas_call(
        _attn_body,
        grid=(n_qt, Hq),
        in_specs=[
            pl.BlockSpec((1, QT, Dm), lambda qt, h: (0, qt, 0)),
            pl.BlockSpec((Dm, Dh), lambda qt, h: (0, h)),
            pl.BlockSpec((Skv, Dh), lambda qt, h: (0, h)),
            pl.BlockSpec((Skv, Dh), lambda qt, h: (0, h)),
            pl.BlockSpec((Dh, Dm), lambda qt, h: (h, 0)),
        ],
        out_specs=pl.BlockSpec((1, QT, Dm), lambda qt, h: (0, qt, 0)),
        out_shape=jax.ShapeDtypeStruct((B, Sq, Dm), jnp.float32),
        compiler_params=pltpu.CompilerParams(
            dimension_semantics=("arbitrary", "arbitrary"),
        ),
    )(x, Wq, kfull, vfull, Wo)
    return out


# device time: 273253 ns/iter; 2.1790x vs baseline; 2.1790x over previous
import jax
import jax.numpy as jnp
from jax import lax
from jax.experimental import pallas as pl
from jax.experimental.pallas import tpu as pltpu

N_DEV = 8
SCALE = 0.08838834764831843
BLK = 64
QT = 512


NC = 8
CH = 256
TOTC = 2 * NC

TREE_CHILDREN = [
    {0: [1], 1: [2, 5], 2: [3, 6], 5: [4], 6: [7]},
    {0: [3], 3: [7, 2], 7: [4, 6], 2: [1], 6: [5]},
    {0: [4], 4: [5, 7], 5: [1, 6], 7: [3], 6: [2]},
]


def _tree_depths():
    depths = []
    for ch in TREE_CHILDREN:
        d = {0: 0}
        frontier = [0]
        while frontier:
            nxt = []
            for u in frontier:
                for v in ch.get(u, []):
                    d[v] = d[u] + 1
                    nxt.append(v)
            frontier = nxt
        depths.append(d)
    return depths


TREE_DEPTH = _tree_depths()


def _chunk_rank(c):
    return c // 3


def _wait_order(p):
    return sorted(
        range(TOTC),
        key=lambda c: (_chunk_rank(c) + TREE_DEPTH[c % 3][p], _chunk_rank(c), c),
    )


def _bcast_body(k_ref, v_ref, kout_ref, vout_ref, r_sems, s0_sems, s1_sems):
    my = lax.axis_index("i")

    def chunk_ref(kb, vb, c):
        base = kb if c < NC else vb
        return base.at[pl.ds((c % NC) * CH, CH), :]

    def desc(c, child, slot, from_input):
        kb = k_ref if from_input else kout_ref
        vb = v_ref if from_input else vout_ref
        sem = s0_sems if slot == 0 else s1_sems
        return pltpu.make_async_remote_copy(
            src_ref=chunk_ref(kb, vb, c),
            dst_ref=chunk_ref(kout_ref, vout_ref, c),
            send_sem=sem.at[c],
            recv_sem=r_sems.at[c],
            device_id=(child,),
            device_id_type=pl.DeviceIdType.MESH,
        )

    @pl.when(my == 0)
    def _():
        for c in range(TOTC):
            for slot, child in enumerate(TREE_CHILDREN[c % 3].get(0, [])):
                desc(c, child, slot, True).start()
        kout_ref[...] = k_ref[...]
        vout_ref[...] = v_ref[...]
        for c in range(TOTC):
            for slot, child in enumerate(TREE_CHILDREN[c % 3].get(0, [])):
                desc(c, child, slot, True).wait_send()

    for p in range(1, N_DEV):

        @pl.when(my == p)
        def _(p=p):
            sends = []
            for c in _wait_order(p):
                desc(c, 0, 0, False).wait_recv()
                for slot, child in enumerate(TREE_CHILDREN[c % 3].get(p, [])):
                    desc(c, child, slot, False).start()
                    sends.append((c, child, slot))
            for c, child, slot in sends:
                desc(c, child, slot, False).wait_send()


def _attn_body(x_ref, wq_ref, k_ref, v_ref, wo_ref, out_ref):
    qt = pl.program_id(0)
    h = pl.program_id(1)

    xm = x_ref[0]
    q = jnp.dot(xm, wq_ref[...], preferred_element_type=jnp.float32)
    k = k_ref[...]
    s = lax.dot_general(
        q, k, (((1,), (1,)), ((), ())), preferred_element_type=jnp.float32
    )
    s = s * SCALE
    row = lax.broadcasted_iota(jnp.int32, s.shape, 0) + qt * QT
    col = lax.broadcasted_iota(jnp.int32, s.shape, 1)
    s = jnp.where((col // BLK) <= (row // BLK), s, -1e9)
    m = jnp.max(s, axis=1, keepdims=True)
    w = jnp.exp(s - m)
    w = w / jnp.sum(w, axis=1, keepdims=True)
    ctx = jnp.dot(w, v_ref[...], preferred_element_type=jnp.float32)
    contrib = jnp.dot(ctx, wo_ref[...], preferred_element_type=jnp.float32)

    @pl.when(h == 0)
    def _():
        out_ref[...] = contrib[None]

    @pl.when(h > 0)
    def _():
        out_ref[...] = out_ref[...] + contrib[None]


def kernel(x, Wq, K_ext, V_ext, Wo):
    B, Sq, Dm = x.shape
    _, Skv, Hq, Dh = K_ext.shape

    k2 = K_ext.reshape(Skv, Hq * Dh)
    v2 = V_ext.reshape(Skv, Hq * Dh)

    kfull, vfull = pl.pallas_call(
        _bcast_body,
        out_shape=[
            jax.ShapeDtypeStruct((Skv, Hq * Dh), jnp.float32),
            jax.ShapeDtypeStruct((Skv, Hq * Dh), jnp.float32),
        ],
        in_specs=[
            pl.BlockSpec(memory_space=pltpu.VMEM),
            pl.BlockSpec(memory_space=pltpu.VMEM),
        ],
        out_specs=[
            pl.BlockSpec(memory_space=pltpu.VMEM),
            pl.BlockSpec(memory_space=pltpu.VMEM),
        ],
        scratch_shapes=[
            pltpu.SemaphoreType.DMA((TOTC,)),
            pltpu.SemaphoreType.DMA((TOTC,)),
            pltpu.SemaphoreType.DMA((TOTC,)),
        ],
    )(k2, v2)

    n_qt = Sq // QT
    out = pl.pallas_call(
        _attn_body,
        grid=(n_qt, Hq),
        in_specs=[
            pl.BlockSpec((1, QT, Dm), lambda qt, h: (0, qt, 0)),
            pl.BlockSpec((Dm, Dh), lambda qt, h: (0, h)),
            pl.BlockSpec((Skv, Dh), lambda qt, h: (0, h)),
            pl.BlockSpec((Skv, Dh), lambda qt, h: (0, h)),
            pl.BlockSpec((Dh, Dm), lambda qt, h: (h, 0)),
        ],
        out_specs=pl.BlockSpec((1, QT, Dm), lambda qt, h: (0, qt, 0)),
        out_shape=jax.ShapeDtypeStruct((B, Sq, Dm), jnp.float32),
        compiler_params=pltpu.CompilerParams(
            dimension_semantics=("arbitrary", "arbitrary"),
        ),
    )(x, Wq, kfull, vfull, Wo)
    return out


# device time: 250267 ns/iter; 2.3792x vs baseline; 1.0918x over previous
import jax
import jax.numpy as jnp
from jax import lax
from jax.experimental import pallas as pl
from jax.experimental.pallas import tpu as pltpu

N_DEV = 8
SCALE = 0.08838834764831843
BLK = 64
QT = 512


NC = 8
CH = 256
TOTC = 2 * NC

TREE_CHILDREN = [
    {0: [1], 1: [2, 5], 2: [3, 6], 5: [4], 6: [7]},
    {0: [3], 3: [7, 2], 7: [4, 6], 2: [1], 6: [5]},
    {0: [4], 4: [5, 7], 5: [1, 6], 7: [3], 6: [2]},
]


def _tree_depths():
    depths = []
    for ch in TREE_CHILDREN:
        d = {0: 0}
        frontier = [0]
        while frontier:
            nxt = []
            for u in frontier:
                for v in ch.get(u, []):
                    d[v] = d[u] + 1
                    nxt.append(v)
            frontier = nxt
        depths.append(d)
    return depths


TREE_DEPTH = _tree_depths()


def _chunk_rank(c):
    return c // 3


def _wait_order(p):
    return sorted(
        range(TOTC),
        key=lambda c: (_chunk_rank(c) + TREE_DEPTH[c % 3][p], _chunk_rank(c), c),
    )


def _bcast_body(k_ref, v_ref, kout_ref, vout_ref, r_sems, s0_sems, s1_sems):
    my = lax.axis_index("i")

    def chunk_ref(kb, vb, c):
        base = kb if c < NC else vb
        return base.at[pl.ds((c % NC) * CH, CH), :]

    def desc(c, child, slot, from_input):
        kb = k_ref if from_input else kout_ref
        vb = v_ref if from_input else vout_ref
        sem = s0_sems if slot == 0 else s1_sems
        return pltpu.make_async_remote_copy(
            src_ref=chunk_ref(kb, vb, c),
            dst_ref=chunk_ref(kout_ref, vout_ref, c),
            send_sem=sem.at[c],
            recv_sem=r_sems.at[c],
            device_id=(child,),
            device_id_type=pl.DeviceIdType.MESH,
        )

    @pl.when(my == 0)
    def _():
        for c in range(TOTC):
            for slot, child in enumerate(TREE_CHILDREN[c % 3].get(0, [])):
                desc(c, child, slot, True).start()
        kout_ref[...] = k_ref[...]
        vout_ref[...] = v_ref[...]
        for c in range(TOTC):
            for slot, child in enumerate(TREE_CHILDREN[c % 3].get(0, [])):
                desc(c, child, slot, True).wait_send()

    for p in range(1, N_DEV):

        @pl.when(my == p)
        def _(p=p):
            sends = []
            for c in _wait_order(p):
                desc(c, 0, 0, False).wait_recv()
                for slot, child in enumerate(TREE_CHILDREN[c % 3].get(p, [])):
                    desc(c, child, slot, False).start()
                    sends.append((c, child, slot))
            for c, child, slot in sends:
                desc(c, child, slot, False).wait_send()


KT = 512
HQ = 8
DH = 128


def _attn_body(x_ref, wq_ref, k_ref, v_ref, wo_ref, out_ref, q_ref, s_ref, ctx_ref):
    qt = pl.program_id(0)
    h = pl.program_id(1)
    nkt = s_ref.shape[1] // KT

    @pl.when(h == 0)
    def _():
        q_ref[...] = jnp.dot(
            x_ref[0], wq_ref[...], preferred_element_type=jnp.float32
        )

    q = q_ref[:, pl.ds(h * DH, DH)]

    for kt in range(nkt):
        sl = pl.ds(kt * KT, KT)

        @pl.when(qt >= kt)
        def _(kt=kt, sl=sl):
            sc = lax.dot_general(
                q,
                k_ref[pl.ds(kt * KT, KT), :],
                (((1,), (1,)), ((), ())),
                preferred_element_type=jnp.float32,
            )
            sc = sc * SCALE
            row = lax.broadcasted_iota(jnp.int32, sc.shape, 0) + qt * QT
            col = lax.broadcasted_iota(jnp.int32, sc.shape, 1) + kt * KT
            s_ref[:, sl] = jnp.where((col // BLK) <= (row // BLK), sc, -1e9)

        @pl.when(qt < kt)
        def _(sl=sl):
            s_ref[:, sl] = jnp.full((s_ref.shape[0], KT), -1e9, jnp.float32)

    s = s_ref[...]
    m = jnp.max(s, axis=1, keepdims=True)
    w = jnp.exp(s - m)
    w = w / jnp.sum(w, axis=1, keepdims=True)
    s_ref[...] = w

    hcols = pl.ds(h * DH, DH)
    ctx_ref[:, hcols] = jnp.dot(
        s_ref[:, pl.ds(0, KT)],
        v_ref[pl.ds(0, KT), :],
        preferred_element_type=jnp.float32,
    )
    for kt in range(1, nkt):

        @pl.when(qt >= kt)
        def _(kt=kt):
            ctx_ref[:, hcols] = ctx_ref[:, hcols] + jnp.dot(
                s_ref[:, pl.ds(kt * KT, KT)],
                v_ref[pl.ds(kt * KT, KT), :],
                preferred_element_type=jnp.float32,
            )

    @pl.when(h == HQ - 1)
    def _():
        out_ref[...] = jnp.dot(
            ctx_ref[...], wo_ref[...], preferred_element_type=jnp.float32
        )[None]


def kernel(x, Wq, K_ext, V_ext, Wo):
    B, Sq, Dm = x.shape
    _, Skv, Hq, Dh = K_ext.shape

    k2 = K_ext.reshape(Skv, Hq * Dh)
    v2 = V_ext.reshape(Skv, Hq * Dh)

    kfull, vfull = pl.pallas_call(
        _bcast_body,
        out_shape=[
            jax.ShapeDtypeStruct((Skv, Hq * Dh), jnp.float32),
            jax.ShapeDtypeStruct((Skv, Hq * Dh), jnp.float32),
        ],
        in_specs=[
            pl.BlockSpec(memory_space=pltpu.VMEM),
            pl.BlockSpec(memory_space=pltpu.VMEM),
        ],
        out_specs=[
            pl.BlockSpec(memory_space=pltpu.VMEM),
            pl.BlockSpec(memory_space=pltpu.VMEM),
        ],
        scratch_shapes=[
            pltpu.SemaphoreType.DMA((TOTC,)),
            pltpu.SemaphoreType.DMA((TOTC,)),
            pltpu.SemaphoreType.DMA((TOTC,)),
        ],
    )(k2, v2)

    n_qt = Sq // QT
    out = pl.pallas_call(
        _attn_body,
        grid=(n_qt, Hq),
        in_specs=[
            pl.BlockSpec((1, QT, Dm), lambda qt, h: (0, qt, 0)),
            pl.BlockSpec((Dm, Dm), lambda qt, h: (0, 0)),
            pl.BlockSpec((Skv, Dh), lambda qt, h: (0, h)),
            pl.BlockSpec((Skv, Dh), lambda qt, h: (0, h)),
            pl.BlockSpec((Dm, Dm), lambda qt, h: (0, 0)),
        ],
        out_specs=pl.BlockSpec((1, QT, Dm), lambda qt, h: (0, qt, 0)),
        out_shape=jax.ShapeDtypeStruct((B, Sq, Dm), jnp.float32),
        scratch_shapes=[
            pltpu.VMEM((QT, Dm), jnp.float32),
            pltpu.VMEM((QT, Skv), jnp.float32),
            pltpu.VMEM((QT, Dm), jnp.float32),
        ],
        compiler_params=pltpu.CompilerParams(
            dimension_semantics=("arbitrary", "arbitrary"),
        ),
    )(x, Wq, kfull, vfull, Wo)
    return out
